# baseline (speedup 1.0000x reference)
import functools

import jax
import jax.numpy as jnp
from jax import lax
from jax.experimental import pallas as pl
from jax.experimental.pallas import tpu as pltpu
from jax.experimental.pallas import tpu_sc as plsc

V = 8
L = 729
N = V * L
C = 1152
NCORE = 2
NSUB = 16
NB = 368
NBLK = 3
CCH = 128
NCHT = C // CCH
NCHC = 5
NV = 4224
DUMMY = 4096
RPT = NV // NSUB
ZR = 24
CB = 16


def _sc_body(video, coord, out, acc, recip_sh, feat0, feat1, coordv, idxv,
             rv, sem, wsem):
    cid = lax.axis_index("c")
    sid = lax.axis_index("s")
    w = sid.astype(jnp.int32)
    own_lo = pl.multiple_of(360 * w + 8 * jnp.minimum(w, 9), 8)
    own_n = jnp.where(w < 9, 368, 360)

    zeros16 = jnp.zeros((CB,), jnp.float32)
    ones16 = jnp.ones((CB,), jnp.float32)
    feats = [feat0, feat1]

    def _stage(fbuf, c0):
        @pl.when(w < 9)
        def _():
            pltpu.make_async_copy(
                video.at[pl.ds(own_lo, 368), pl.ds(c0, CCH)],
                fbuf.at[pl.ds(0, 368)], sem).start()

        @pl.when(w >= 9)
        def _():
            pltpu.make_async_copy(
                video.at[pl.ds(own_lo, 360), pl.ds(c0, CCH)],
                fbuf.at[pl.ds(0, 360)], sem).start()

    def _stage_wait(fbuf, c0):
        @pl.when(w < 9)
        def _():
            pltpu.make_async_copy(
                video.at[pl.ds(own_lo, 368), pl.ds(c0, CCH)],
                fbuf.at[pl.ds(0, 368)], sem).wait()

        @pl.when(w >= 9)
        def _():
            pltpu.make_async_copy(
                video.at[pl.ds(own_lo, 360), pl.ds(c0, CCH)],
                fbuf.at[pl.ds(0, 360)], sem).wait()

    def _c0_of(c_idx):
        return pl.multiple_of(c_idx * CCH, CCH)

    def _wb_descs(fbuf, c0):
        return [
            pltpu.make_async_copy(
                fbuf.at[pl.ds(0, 128)],
                out.at[pl.ds(own_lo, 128), pl.ds(c0, CCH)], wsem),
            pltpu.make_async_copy(
                fbuf.at[pl.ds(128, 128)],
                out.at[pl.ds(pl.multiple_of(own_lo + 128, 8), 128),
                       pl.ds(c0, CCH)], wsem),
            pltpu.make_async_copy(
                fbuf.at[pl.ds(base3, 128)],
                out.at[pl.ds(pl.multiple_of(own_lo + base3, 8), 128),
                       pl.ds(c0, CCH)], wsem),
        ]

    _stage(feat0, _c0_of(cid * NCHC))

    cbase = pl.multiple_of(own_lo * 3, 8)
    pltpu.sync_copy(coord.at[pl.ds(cbase, 1104)], coordv.at[pl.ds(0, 1104)])
    base3 = pl.multiple_of(own_n - 128, 8)

    def _vox(rows):
        fl = rows * 3
        gx = plsc.load_gather(coordv, [fl])
        gy = plsc.load_gather(coordv, [fl + 1])
        gz = plsc.load_gather(coordv, [fl + 2])
        return (gx.astype(jnp.int32) * 256 + gy.astype(jnp.int32) * 16
                + gz.astype(jnp.int32))

    def _idx01(j, carry):
        rows = lax.iota(jnp.int32, CB) + (j * CB)
        col = (j % 8) * CB
        @pl.when(j < 8)
        def _():
            idxv[0, pl.ds(col, CB)] = _vox(rows)
        @pl.when(j >= 8)
        def _():
            idxv[1, pl.ds(col, CB)] = _vox(rows)
        return carry

    lax.fori_loop(0, 16, _idx01, 0)

    def _idx23(jb, carry):
        rows = lax.iota(jnp.int32, CB) + jb * CB + base3
        vox = _vox(rows)
        idxv[2, pl.ds(jb * CB, CB)] = jnp.where(rows < 256, DUMMY, vox)
        idxv[3, pl.ds(jb * CB, CB)] = vox
        return carry

    lax.fori_loop(0, 8, _idx23, 0)

    def _zero_acc(buf):
        def _zfill(i, carry):
            for k in range(CCH // CB):
                buf[i, pl.ds(k * CB, CB)] = zeros16
            return carry

        lax.fori_loop(0, ZR, _zfill, 0)
        for b in range(RPT // ZR):
            pltpu.sync_copy(buf.at[pl.ds(0, ZR)],
                            acc.at[pl.ds(w * RPT + b * ZR, ZR)])

    _zero_acc(feat1)

    def _ofill(i, carry):
        for k in range(CCH // CB):
            feat1[i, pl.ds(k * CB, CB)] = ones16
        return carry

    lax.fori_loop(0, 128, _ofill, 0)
    plsc.subcore_barrier()
    for jj in range(NBLK):
        pltpu.sync_copy(feat1.at[pl.ds(0, 128)], acc.at[idxv.at[jj]],
                        add=True)
    plsc.subcore_barrier()

    pltpu.sync_copy(acc.at[pl.ds(w * RPT, RPT)], feat1.at[pl.ds(0, RPT)])

    def _recip(k, carry):
        rows = lax.iota(jnp.int32, CB) + (k * CB)
        c16 = plsc.load_gather(feat1, [rows, jnp.zeros((CB,), jnp.int32)])
        rv[pl.ds(k * CB, CB)] = 1.0 / jnp.maximum(c16, 1.0)
        return carry

    lax.fori_loop(0, (RPT + CB - 1) // CB, _recip, 0)
    pltpu.sync_copy(rv.at[pl.ds(0, RPT)], recip_sh.at[pl.ds(w * RPT, RPT)])
    plsc.subcore_barrier()
    pltpu.sync_copy(recip_sh.at[idxv.at[0]], rv.at[pl.ds(0, 128)])
    pltpu.sync_copy(recip_sh.at[idxv.at[1]], rv.at[pl.ds(128, 128)])
    pltpu.sync_copy(recip_sh.at[idxv.at[3]], rv.at[pl.ds(base3, 128)])

    for ch in range(NCHC):
        c_idx = cid * NCHC + ch
        cur = feats[ch % 2]
        nxt = feats[(ch + 1) % 2]

        @pl.when(c_idx < NCHT)
        def _(c_idx=c_idx, ch=ch, cur=cur, nxt=nxt):
            c0 = _c0_of(c_idx)
            if ch > 0:
                for d in _wb_descs(nxt, _c0_of(c_idx - 1)):
                    d.wait()
            _zero_acc(nxt)
            if ch + 1 < NCHC:
                @pl.when(c_idx + 1 < NCHT)
                def _():
                    _stage(nxt, _c0_of(c_idx + 1))
            _stage_wait(cur, c0)
            plsc.subcore_barrier()

            pltpu.sync_copy(cur.at[pl.ds(0, 128)],
                            acc.at[idxv.at[0]], add=True)
            pltpu.sync_copy(cur.at[pl.ds(128, 128)],
                            acc.at[idxv.at[1]], add=True)
            pltpu.sync_copy(cur.at[pl.ds(base3, 128)],
                            acc.at[idxv.at[2]], add=True)
            plsc.subcore_barrier()

            pltpu.sync_copy(acc.at[idxv.at[0]], cur.at[pl.ds(0, 128)])
            pltpu.sync_copy(acc.at[idxv.at[1]], cur.at[pl.ds(128, 128)])
            pltpu.sync_copy(acc.at[idxv.at[3]], cur.at[pl.ds(base3, 128)])
            plsc.subcore_barrier()

            def _scale_row(i, carry):
                rs = plsc.load_gather(rv, [jnp.zeros((CB,), jnp.int32) + i])
                for k in range(CCH // CB):
                    cur[i, pl.ds(k * CB, CB)] = cur[i, pl.ds(k * CB, CB)] * rs
                return carry

            wb = _wb_descs(cur, c0)
            lax.fori_loop(0, 128, _scale_row, 0)
            wb[0].start()
            lax.fori_loop(128, 256, _scale_row, 0)
            wb[1].start()
            lax.fori_loop(256, own_n, _scale_row, 0)
            wb[2].start()

    @pl.when(cid == 0)
    def _():
        for d in _wb_descs(feats[(NCHC - 1) % 2], _c0_of(NCHC - 1)):
            d.wait()

    @pl.when(cid == 1)
    def _():
        for d in _wb_descs(feats[(NCHC - 2) % 2], _c0_of(NCHT - 1)):
            d.wait()


_MESH = plsc.VectorSubcoreMesh(core_axis_name="c", subcore_axis_name="s",
                               num_cores=NCORE, num_subcores=NSUB)

_sc_call = functools.partial(
    pl.kernel,
    out_type=jax.ShapeDtypeStruct((N, C), jnp.float32),
    mesh=_MESH,
    compiler_params=pltpu.CompilerParams(needs_layout_passes=False,
                                         use_tc_tiling_on_sc=True),
    scratch_types=[
        pltpu.MemorySpace.VMEM_SHARED((NV, CCH), jnp.float32),
        pltpu.MemorySpace.VMEM_SHARED((NV,), jnp.float32),
        pltpu.VMEM((NB, CCH), jnp.float32),
        pltpu.VMEM((NB, CCH), jnp.float32),
        pltpu.VMEM((NB * 3,), jnp.float32),
        pltpu.VMEM((4, 128), jnp.int32),
        pltpu.VMEM((NB,), jnp.float32),
        pltpu.SemaphoreType.DMA,
        pltpu.SemaphoreType.DMA,
    ],
)(_sc_body)


def kernel(video_tensor, coord_info):
    v2 = jnp.transpose(video_tensor, (1, 0, 2)).reshape(N, C)
    cflat = jnp.transpose(coord_info, (1, 2, 0, 3)).reshape(N * 3)
    cflat = jnp.pad(cflat, (0, 24))
    out = _sc_call(v2, cflat)
    return jnp.transpose(out.reshape(L, V, C), (1, 0, 2))


# speedup vs baseline: 1.0193x; 1.0193x over previous
import functools

import jax
import jax.numpy as jnp
from jax import lax
from jax.experimental import pallas as pl
from jax.experimental.pallas import tpu as pltpu
from jax.experimental.pallas import tpu_sc as plsc

V = 8
L = 729
N = V * L
C = 1152
NCORE = 2
NSUB = 16
NB = 368
NBLK = 3
CCH = 128
NCHT = C // CCH
NCHC = 5
NV = 4224
DUMMY = 4096
RPT = NV // NSUB
ZR = 24
CB = 16


def _sc_body(video, coord, out, acc, recip_sh, feat0, feat1, coordv, idxv,
             rv, sem, wsem):
    cid = lax.axis_index("c")
    sid = lax.axis_index("s")
    w = sid.astype(jnp.int32)
    own_lo = pl.multiple_of(360 * w + 8 * jnp.minimum(w, 9), 8)
    own_n = jnp.where(w < 9, 368, 360)

    zeros16 = jnp.zeros((CB,), jnp.float32)
    ones16 = jnp.ones((CB,), jnp.float32)
    feats = [feat0, feat1]

    def _stage(fbuf, c0):
        @pl.when(w < 9)
        def _():
            pltpu.make_async_copy(
                video.at[pl.ds(own_lo, 368), pl.ds(c0, CCH)],
                fbuf.at[pl.ds(0, 368)], sem).start()

        @pl.when(w >= 9)
        def _():
            pltpu.make_async_copy(
                video.at[pl.ds(own_lo, 360), pl.ds(c0, CCH)],
                fbuf.at[pl.ds(0, 360)], sem).start()

    def _stage_wait(fbuf, c0):
        @pl.when(w < 9)
        def _():
            pltpu.make_async_copy(
                video.at[pl.ds(own_lo, 368), pl.ds(c0, CCH)],
                fbuf.at[pl.ds(0, 368)], sem).wait()

        @pl.when(w >= 9)
        def _():
            pltpu.make_async_copy(
                video.at[pl.ds(own_lo, 360), pl.ds(c0, CCH)],
                fbuf.at[pl.ds(0, 360)], sem).wait()

    def _c0_of(c_idx):
        return pl.multiple_of(c_idx * CCH, CCH)

    def _wb_descs(fbuf, c0):
        return [
            pltpu.make_async_copy(
                fbuf.at[pl.ds(0, 128)],
                out.at[pl.ds(own_lo, 128), pl.ds(c0, CCH)], wsem),
            pltpu.make_async_copy(
                fbuf.at[pl.ds(128, 128)],
                out.at[pl.ds(pl.multiple_of(own_lo + 128, 8), 128),
                       pl.ds(c0, CCH)], wsem),
            pltpu.make_async_copy(
                fbuf.at[pl.ds(base3, 128)],
                out.at[pl.ds(pl.multiple_of(own_lo + base3, 8), 128),
                       pl.ds(c0, CCH)], wsem),
        ]

    _stage(feat0, _c0_of(cid * NCHC))

    cbase = pl.multiple_of(own_lo * 3, 8)
    pltpu.sync_copy(coord.at[pl.ds(cbase, 1104)], coordv.at[pl.ds(0, 1104)])
    base3 = pl.multiple_of(own_n - 128, 8)

    def _vox(rows):
        fl = rows * 3
        gx = plsc.load_gather(coordv, [fl])
        gy = plsc.load_gather(coordv, [fl + 1])
        gz = plsc.load_gather(coordv, [fl + 2])
        return (gx.astype(jnp.int32) * 256 + gy.astype(jnp.int32) * 16
                + gz.astype(jnp.int32))

    def _idx01(j, carry):
        rows = lax.iota(jnp.int32, CB) + (j * CB)
        col = (j % 8) * CB
        @pl.when(j < 8)
        def _():
            idxv[0, pl.ds(col, CB)] = _vox(rows)
        @pl.when(j >= 8)
        def _():
            idxv[1, pl.ds(col, CB)] = _vox(rows)
        return carry

    lax.fori_loop(0, 16, _idx01, 0)

    def _idx23(jb, carry):
        rows = lax.iota(jnp.int32, CB) + jb * CB + base3
        vox = _vox(rows)
        idxv[2, pl.ds(jb * CB, CB)] = jnp.where(rows < 256, DUMMY, vox)
        idxv[3, pl.ds(jb * CB, CB)] = vox
        return carry

    lax.fori_loop(0, 8, _idx23, 0)

    def _zero_acc(buf):
        def _zfill(i, carry):
            for k in range(CCH // CB):
                buf[i, pl.ds(k * CB, CB)] = zeros16
            return carry

        lax.fori_loop(0, ZR, _zfill, 0)
        for b in range(RPT // ZR):
            pltpu.sync_copy(buf.at[pl.ds(0, ZR)],
                            acc.at[pl.ds(w * RPT + b * ZR, ZR)])

    _zero_acc(feat1)

    def _ofill(i, carry):
        for k in range(CCH // CB):
            feat1[i, pl.ds(k * CB, CB)] = ones16
        return carry

    lax.fori_loop(0, 128, _ofill, 0)
    plsc.subcore_barrier()
    for jj in range(NBLK):
        pltpu.sync_copy(feat1.at[pl.ds(0, 128)], acc.at[idxv.at[jj]],
                        add=True)
    plsc.subcore_barrier()

    pltpu.sync_copy(acc.at[pl.ds(w * RPT, RPT)], feat1.at[pl.ds(0, RPT)])

    def _recip(k, carry):
        rows = lax.iota(jnp.int32, CB) + (k * CB)
        c16 = plsc.load_gather(feat1, [rows, jnp.zeros((CB,), jnp.int32)])
        rv[pl.ds(k * CB, CB)] = 1.0 / jnp.maximum(c16, 1.0)
        return carry

    lax.fori_loop(0, (RPT + CB - 1) // CB, _recip, 0)
    pltpu.sync_copy(rv.at[pl.ds(0, RPT)], recip_sh.at[pl.ds(w * RPT, RPT)])
    plsc.subcore_barrier()
    pltpu.sync_copy(recip_sh.at[idxv.at[0]], rv.at[pl.ds(0, 128)])
    pltpu.sync_copy(recip_sh.at[idxv.at[1]], rv.at[pl.ds(128, 128)])
    pltpu.sync_copy(recip_sh.at[idxv.at[3]], rv.at[pl.ds(base3, 128)])

    for ch in range(NCHC):
        c_idx = cid * NCHC + ch
        cur = feats[ch % 2]
        nxt = feats[(ch + 1) % 2]

        @pl.when(c_idx < NCHT)
        def _(c_idx=c_idx, ch=ch, cur=cur, nxt=nxt):
            c0 = _c0_of(c_idx)
            if ch > 0:
                for d in _wb_descs(nxt, _c0_of(c_idx - 1)):
                    d.wait()
            _zero_acc(nxt)
            _stage_wait(cur, c0)
            plsc.subcore_barrier()

            pltpu.sync_copy(cur.at[pl.ds(0, 128)],
                            acc.at[idxv.at[0]], add=True)
            pltpu.sync_copy(cur.at[pl.ds(128, 128)],
                            acc.at[idxv.at[1]], add=True)
            pltpu.sync_copy(cur.at[pl.ds(base3, 128)],
                            acc.at[idxv.at[2]], add=True)
            plsc.subcore_barrier()

            if ch + 1 < NCHC:
                @pl.when(c_idx + 1 < NCHT)
                def _():
                    _stage(nxt, _c0_of(c_idx + 1))

            pltpu.sync_copy(acc.at[idxv.at[0]], cur.at[pl.ds(0, 128)])
            pltpu.sync_copy(acc.at[idxv.at[1]], cur.at[pl.ds(128, 128)])
            pltpu.sync_copy(acc.at[idxv.at[3]], cur.at[pl.ds(base3, 128)])
            plsc.subcore_barrier()

            def _scale_row(i, carry):
                rs = plsc.load_gather(rv, [jnp.zeros((CB,), jnp.int32) + i])
                for k in range(CCH // CB):
                    cur[i, pl.ds(k * CB, CB)] = cur[i, pl.ds(k * CB, CB)] * rs
                return carry

            wb = _wb_descs(cur, c0)
            lax.fori_loop(0, 128, _scale_row, 0)
            wb[0].start()
            lax.fori_loop(128, 256, _scale_row, 0)
            wb[1].start()
            lax.fori_loop(256, own_n, _scale_row, 0)
            wb[2].start()

    @pl.when(cid == 0)
    def _():
        for d in _wb_descs(feats[(NCHC - 1) % 2], _c0_of(NCHC - 1)):
            d.wait()

    @pl.when(cid == 1)
    def _():
        for d in _wb_descs(feats[(NCHC - 2) % 2], _c0_of(NCHT - 1)):
            d.wait()


_MESH = plsc.VectorSubcoreMesh(core_axis_name="c", subcore_axis_name="s",
                               num_cores=NCORE, num_subcores=NSUB)

_sc_call = functools.partial(
    pl.kernel,
    out_type=jax.ShapeDtypeStruct((N, C), jnp.float32),
    mesh=_MESH,
    compiler_params=pltpu.CompilerParams(needs_layout_passes=False,
                                         use_tc_tiling_on_sc=True),
    scratch_types=[
        pltpu.MemorySpace.VMEM_SHARED((NV, CCH), jnp.float32),
        pltpu.MemorySpace.VMEM_SHARED((NV,), jnp.float32),
        pltpu.VMEM((NB, CCH), jnp.float32),
        pltpu.VMEM((NB, CCH), jnp.float32),
        pltpu.VMEM((NB * 3,), jnp.float32),
        pltpu.VMEM((4, 128), jnp.int32),
        pltpu.VMEM((NB,), jnp.float32),
        pltpu.SemaphoreType.DMA,
        pltpu.SemaphoreType.DMA,
    ],
)(_sc_body)


def kernel(video_tensor, coord_info):
    v2 = jnp.transpose(video_tensor, (1, 0, 2)).reshape(N, C)
    cflat = jnp.transpose(coord_info, (1, 2, 0, 3)).reshape(N * 3)
    cflat = jnp.pad(cflat, (0, 24))
    out = _sc_call(v2, cflat)
    return jnp.transpose(out.reshape(L, V, C), (1, 0, 2))
